# baseline (speedup 1.0000x reference)
import jax
import jax.numpy as jnp
from jax.experimental import pallas as pl
from jax.experimental.pallas import tpu as pltpu

B, S = 4096, 50
N_GATE_TYPES = 32
N_QUBITS = 64
D = 128
T = B * S
TB = 1024


def _body(g_ref, q_ref, w_ref, qt_ref, o_ref):
    g = g_ref[...]
    q = q_ref[...]
    w = w_ref[...]
    qt = qt_ref[...]
    emb = jnp.dot(g, w, preferred_element_type=jnp.float32)
    qc = jnp.dot(q[:, :N_QUBITS], qt, preferred_element_type=jnp.float32)
    qt2 = jnp.dot(q[:, N_QUBITS:], qt, preferred_element_type=jnp.float32)
    o_ref[...] = emb + jnp.concatenate([qc, qt2], axis=1)


def kernel(gates_oh, gate_qubits_oh, qubits, W_gate):
    g2 = gates_oh.reshape(T, N_GATE_TYPES)
    q2 = gate_qubits_oh.reshape(T, 2 * N_QUBITS)
    grid = (T // TB,)
    out = pl.pallas_call(
        _body,
        grid=grid,
        in_specs=[
            pl.BlockSpec((TB, N_GATE_TYPES), lambda i: (i, 0)),
            pl.BlockSpec((TB, 2 * N_QUBITS), lambda i: (i, 0)),
            pl.BlockSpec((N_GATE_TYPES, 2 * D), lambda i: (0, 0)),
            pl.BlockSpec((N_QUBITS, D), lambda i: (0, 0)),
        ],
        out_specs=pl.BlockSpec((TB, 2 * D), lambda i: (i, 0)),
        out_shape=jax.ShapeDtypeStruct((T, 2 * D), jnp.float32),
        compiler_params=pltpu.CompilerParams(
            dimension_semantics=("arbitrary",),
        ),
    )(g2, q2, W_gate, qubits)
    return out.reshape(B, S, 2 * D)
